# Initial kernel scaffold; baseline (speedup 1.0000x reference)
import jax
import jax.numpy as jnp
from jax.experimental import pallas as pl


def kernel(x, edge_index, W_self1, W_neigh1, b1, W_self2, W_neigh2, b2):
    raise NotImplementedError("write your pallas kernel here")


# speedup vs baseline: 9.3723x; 9.3723x over previous
import functools

import jax
import jax.numpy as jnp
from jax import lax
from jax.experimental import pallas as pl
from jax.experimental.pallas import tpu as pltpu
from jax.experimental.pallas import tpu_sc as plsc

N = 50000
NP = 53248
RB = 1024
NG = NP // RB
NC = 2
NS = 16
LB = 128
NW = NC * NS
NB = 196
CH = 14
EPW = NB * LB
EP = NW * EPW
E = 800000
TPR = NP // NS
TNB = TPR // LB


def _mm1_body(x_ref, ws_ref, wn_ref, b_ref, s_ref, y_ref):
    xb = x_ref[...]
    s_ref[...] = jnp.dot(xb, ws_ref[...], preferred_element_type=jnp.float32) + b_ref[...]
    y_ref[...] = jnp.dot(xb, wn_ref[...], preferred_element_type=jnp.float32)


def _mid_body(s_ref, agg_ref, deg_ref, ws_ref, wn_ref, b_ref, s2_ref, y_ref):
    a = agg_ref[...]
    d = deg_ref[...]
    rd = 1.0 / jnp.maximum(d[0] + d[1], 1.0)
    h = s_ref[...] + (a[0] + a[1]) * rd
    s2_ref[...] = jnp.dot(h, ws_ref[...], preferred_element_type=jnp.float32) + b_ref[...]
    y_ref[...] = jnp.dot(h, wn_ref[...], preferred_element_type=jnp.float32)


def _out_body(s_ref, agg_ref, deg_ref, o_ref):
    a = agg_ref[...]
    d = deg_ref[...]
    rd = 1.0 / jnp.maximum(d[0] + d[1], 1.0)
    o_ref[...] = s_ref[...] + (a[0] + a[1]) * rd


def _mm1(x_p, ws, wn, b):
    return pl.pallas_call(
        _mm1_body,
        grid=(NG,),
        in_specs=[
            pl.BlockSpec((RB, 64), lambda i: (i, 0)),
            pl.BlockSpec((64, 32), lambda i: (0, 0)),
            pl.BlockSpec((64, 32), lambda i: (0, 0)),
            pl.BlockSpec((1, 32), lambda i: (0, 0)),
        ],
        out_specs=[
            pl.BlockSpec((RB, 32), lambda i: (i, 0)),
            pl.BlockSpec((RB, 32), lambda i: (i, 0)),
        ],
        out_shape=[
            jax.ShapeDtypeStruct((NP, 32), jnp.float32),
            jax.ShapeDtypeStruct((NP, 32), jnp.float32),
        ],
    )(x_p, ws, wn, b)


def _mid(s1, aggp, degp, ws, wn, b):
    return pl.pallas_call(
        _mid_body,
        grid=(NG,),
        in_specs=[
            pl.BlockSpec((RB, 32), lambda i: (i, 0)),
            pl.BlockSpec((NC, RB, 32), lambda i: (0, i, 0)),
            pl.BlockSpec((NC, RB, 1), lambda i: (0, i, 0)),
            pl.BlockSpec((32, 16), lambda i: (0, 0)),
            pl.BlockSpec((32, 16), lambda i: (0, 0)),
            pl.BlockSpec((1, 16), lambda i: (0, 0)),
        ],
        out_specs=[
            pl.BlockSpec((RB, 16), lambda i: (i, 0)),
            pl.BlockSpec((RB, 16), lambda i: (i, 0)),
        ],
        out_shape=[
            jax.ShapeDtypeStruct((NP, 16), jnp.float32),
            jax.ShapeDtypeStruct((NP, 16), jnp.float32),
        ],
    )(s1, aggp, degp, ws, wn, b)


def _out(s2, aggp, degp):
    return pl.pallas_call(
        _out_body,
        grid=(NG,),
        in_specs=[
            pl.BlockSpec((RB, 16), lambda i: (i, 0)),
            pl.BlockSpec((NC, RB, 16), lambda i: (0, i, 0)),
            pl.BlockSpec((NC, RB, 1), lambda i: (0, i, 0)),
        ],
        out_specs=pl.BlockSpec((RB, 16), lambda i: (i, 0)),
        out_shape=jax.ShapeDtypeStruct((NP, 16), jnp.float32),
    )(s2, aggp, degp)


def _make_sc(F, with_deg):
    mesh = plsc.VectorSubcoreMesh(core_axis_name="c", subcore_axis_name="s")
    out_type = [jax.ShapeDtypeStruct((NC, NP, F), jnp.float32)]
    scratch = [
        pltpu.VMEM((CH, LB), jnp.int32),
        pltpu.VMEM((CH, LB), jnp.int32),
        pltpu.VMEM((LB, F), jnp.float32),
        pltpu.VMEM((LB, F), jnp.float32),
        pltpu.SemaphoreType.DMA,
        pltpu.VMEM_SHARED((NP, F), jnp.float32),
    ]
    if with_deg:
        out_type.append(jax.ShapeDtypeStruct((NC, NP), jnp.float32))
        scratch += [
            pltpu.VMEM((LB,), jnp.float32),
            pltpu.VMEM((LB,), jnp.float32),
            pltpu.VMEM_SHARED((NP,), jnp.float32),
        ]

    def body(y_hbm, src_hbm, dst_hbm, *rest):
        if with_deg:
            (agg_out, deg_out, src_v, dst_v, rows_v, zrow_v, sem, agg_sh,
             ones_v, zd_v, deg_sh) = rest
        else:
            agg_out, src_v, dst_v, rows_v, zrow_v, sem, agg_sh = rest
        c = lax.axis_index("c")
        s = lax.axis_index("s")
        w = c * NS + s
        base = s * TPR

        def zrow_init(i, _):
            r = i // (F // 16)
            k = (i % (F // 16)) * 16
            zrow_v[r, pl.ds(k, 16)] = jnp.zeros((16,), jnp.float32)
            return 0
        lax.fori_loop(0, LB * (F // 16), zrow_init, 0)
        if with_deg:
            def vec_init(i, _):
                ones_v[pl.ds(i * 16, 16)] = jnp.full((16,), 1.0, jnp.float32)
                zd_v[pl.ds(i * 16, 16)] = jnp.zeros((16,), jnp.float32)
                return 0
            lax.fori_loop(0, LB // 16, vec_init, 0)

        def zchunk(i, _):
            pltpu.sync_copy(zrow_v, agg_sh.at[pl.ds(base + i * LB, LB)])
            if with_deg:
                pltpu.sync_copy(zd_v, deg_sh.at[pl.ds(base + i * LB, LB)])
            return 0
        lax.fori_loop(0, TNB, zchunk, 0)

        plsc.subcore_barrier()

        def echunk(ci, _):
            pltpu.sync_copy(src_hbm.at[w, pl.ds(ci * CH, CH)], src_v)
            pltpu.sync_copy(dst_hbm.at[w, pl.ds(ci * CH, CH)], dst_v)

            def ebatch(j, _):
                pltpu.async_copy(y_hbm.at[src_v.at[j]], rows_v, sem).wait()
                pltpu.sync_copy(rows_v, agg_sh.at[dst_v.at[j]], add=True)
                if with_deg:
                    pltpu.sync_copy(ones_v, deg_sh.at[dst_v.at[j]], add=True)
                return 0
            lax.fori_loop(0, CH, ebatch, 0)
            return 0
        lax.fori_loop(0, NB // CH, echunk, 0)
        plsc.subcore_barrier()

        def wchunk(i, _):
            off = base + i * LB
            pltpu.sync_copy(agg_sh.at[pl.ds(off, LB)], zrow_v)
            pltpu.sync_copy(zrow_v, agg_out.at[c, pl.ds(off, LB)])
            if with_deg:
                pltpu.sync_copy(deg_sh.at[pl.ds(off, LB)], ones_v)
                pltpu.sync_copy(ones_v, deg_out.at[c, pl.ds(off, LB)])
            return 0
        lax.fori_loop(0, TNB, wchunk, 0)

    return pl.kernel(
        body, out_type=out_type, mesh=mesh, scratch_types=scratch,
        compiler_params=pltpu.CompilerParams(use_tc_tiling_on_sc=False),
    )


_sc32 = _make_sc(32, True)
_sc16 = _make_sc(16, False)


def kernel(x, edge_index, W_self1, W_neigh1, b1, W_self2, W_neigh2, b2):
    src = edge_index[0].astype(jnp.int32)
    dst = edge_index[1].astype(jnp.int32)
    src_p = jnp.concatenate([src, jnp.zeros((EP - E,), jnp.int32)]).reshape(NW, NB, LB)
    dst_p = jnp.concatenate([dst, jnp.full((EP - E,), N, jnp.int32)]).reshape(NW, NB, LB)
    x_p = jnp.pad(x, ((0, NP - N), (0, 0)))

    s1, y1 = _mm1(x_p, W_self1, W_neigh1, b1.reshape(1, 32))
    agg1, deg = _sc32(y1, src_p, dst_p)
    degp = deg.reshape(NC, NP, 1)
    s2, y2 = _mid(s1, agg1, degp, W_self2, W_neigh2, b2.reshape(1, 16))
    agg2 = _sc16(y2, src_p, dst_p)[0]
    out = _out(s2, agg2, degp)
    return out[:N]


# speedup vs baseline: 11.0345x; 1.1774x over previous
import jax
import jax.numpy as jnp
from jax import lax
from jax.experimental import pallas as pl
from jax.experimental.pallas import tpu as pltpu
from jax.experimental.pallas import tpu_sc as plsc

N = 50000
NP = 53248
RB = 1000
NG = N // RB
NC = 2
NS = 16
LB = 128
K = 4
NW = NC * NS
NB = 196
NCH = NB // K
EPW = NB * LB
EP = NW * EPW
E = 800000
TPR = NP // NS
TFC = TPR // LB


def _mm1_body(x_ref, w_ref, b_ref, s_ref, y_ref):
    r = jnp.dot(x_ref[...], w_ref[...], preferred_element_type=jnp.float32) + b_ref[...]
    s_ref[...] = r[:, :32]
    y_ref[...] = r[:, 32:]


def _mid_body(s_ref, agg_ref, deg_ref, w_ref, b_ref, s2_ref, y_ref):
    a = agg_ref[...]
    d = deg_ref[...]
    rd = 1.0 / jnp.maximum(d[0] + d[1], 1.0)
    h = s_ref[...] + (a[0] + a[1]) * rd
    r = jnp.dot(h, w_ref[...], preferred_element_type=jnp.float32) + b_ref[...]
    s2_ref[...] = r[:, :16]
    y_ref[...] = r[:, 16:]


def _out_body(s_ref, agg_ref, deg_ref, o_ref):
    a = agg_ref[...]
    d = deg_ref[...]
    rd = 1.0 / jnp.maximum(d[0] + d[1], 1.0)
    o_ref[...] = s_ref[...] + (a[0] + a[1]) * rd


def _mm1(x, w, b):
    return pl.pallas_call(
        _mm1_body,
        grid=(NG,),
        in_specs=[
            pl.BlockSpec((RB, 64), lambda i: (i, 0)),
            pl.BlockSpec((64, 64), lambda i: (0, 0)),
            pl.BlockSpec((1, 64), lambda i: (0, 0)),
        ],
        out_specs=[
            pl.BlockSpec((RB, 32), lambda i: (i, 0)),
            pl.BlockSpec((RB, 32), lambda i: (i, 0)),
        ],
        out_shape=[
            jax.ShapeDtypeStruct((N, 32), jnp.float32),
            jax.ShapeDtypeStruct((N, 32), jnp.float32),
        ],
    )(x, w, b)


def _mid(s1, aggp, degp, w, b):
    return pl.pallas_call(
        _mid_body,
        grid=(NG,),
        in_specs=[
            pl.BlockSpec((RB, 32), lambda i: (i, 0)),
            pl.BlockSpec((NC, RB, 32), lambda i: (0, i, 0)),
            pl.BlockSpec((NC, RB, 1), lambda i: (0, i, 0)),
            pl.BlockSpec((32, 32), lambda i: (0, 0)),
            pl.BlockSpec((1, 32), lambda i: (0, 0)),
        ],
        out_specs=[
            pl.BlockSpec((RB, 16), lambda i: (i, 0)),
            pl.BlockSpec((RB, 16), lambda i: (i, 0)),
        ],
        out_shape=[
            jax.ShapeDtypeStruct((N, 16), jnp.float32),
            jax.ShapeDtypeStruct((N, 16), jnp.float32),
        ],
    )(s1, aggp, degp, w, b)


def _out(s2, aggp, degp):
    return pl.pallas_call(
        _out_body,
        grid=(NG,),
        in_specs=[
            pl.BlockSpec((RB, 16), lambda i: (i, 0)),
            pl.BlockSpec((NC, RB, 16), lambda i: (0, i, 0)),
            pl.BlockSpec((NC, RB, 1), lambda i: (0, i, 0)),
        ],
        out_specs=pl.BlockSpec((RB, 16), lambda i: (i, 0)),
        out_shape=jax.ShapeDtypeStruct((N, 16), jnp.float32),
    )(s2, aggp, degp)


def _make_sc(F, with_deg):
    mesh = plsc.VectorSubcoreMesh(core_axis_name="c", subcore_axis_name="s")
    out_type = [jax.ShapeDtypeStruct((NC, NP, F), jnp.float32)]
    scratch = [
        pltpu.VMEM((K, LB), jnp.int32),
        pltpu.VMEM((K, LB), jnp.int32),
        pltpu.VMEM((K, LB, F), jnp.float32),
        [pltpu.SemaphoreType.DMA] * K,
        pltpu.VMEM_SHARED((NP, F), jnp.float32),
    ]
    if with_deg:
        out_type.append(jax.ShapeDtypeStruct((NC, NP), jnp.float32))
        scratch += [
            pltpu.VMEM((LB,), jnp.float32),
            pltpu.VMEM((LB,), jnp.float32),
            pltpu.VMEM_SHARED((NP,), jnp.float32),
        ]

    def body(y_hbm, src_hbm, dst_hbm, zrow_hbm, ones_hbm, z1_hbm, *rest):
        if with_deg:
            (agg_out, deg_out, src_v, dst_v, rows_v, gsem, agg_sh,
             ones_v, zd_v, deg_sh) = rest
        else:
            agg_out, src_v, dst_v, rows_v, gsem, agg_sh = rest
        c = lax.axis_index("c")
        s = lax.axis_index("s")
        w = c * NS + s
        base = s * TPR

        pltpu.sync_copy(zrow_hbm, rows_v.at[0])
        if with_deg:
            pltpu.sync_copy(ones_hbm, ones_v)
            pltpu.sync_copy(z1_hbm, zd_v)

        def zchunk(i, _):
            pltpu.sync_copy(rows_v.at[0], agg_sh.at[pl.ds(base + i * LB, LB)])
            if with_deg:
                pltpu.sync_copy(zd_v, deg_sh.at[pl.ds(base + i * LB, LB)])
            return 0
        lax.fori_loop(0, TFC, zchunk, 0)
        plsc.subcore_barrier()

        def chunk(ci, _):
            s0 = ci * K
            pltpu.sync_copy(src_hbm.at[w, pl.ds(s0, K)], src_v)
            pltpu.sync_copy(dst_hbm.at[w, pl.ds(s0, K)], dst_v)
            gds = [pltpu.async_copy(y_hbm.at[src_v.at[b]], rows_v.at[b], gsem[b])
                   for b in range(K)]
            for b in range(K):
                gds[b].wait()
                pltpu.sync_copy(rows_v.at[b], agg_sh.at[dst_v.at[b]], add=True)
                if with_deg:
                    pltpu.sync_copy(ones_v, deg_sh.at[dst_v.at[b]], add=True)
            return 0
        lax.fori_loop(0, NCH, chunk, 0)
        plsc.subcore_barrier()

        def wchunk(i, _):
            off = base + i * LB
            pltpu.sync_copy(agg_sh.at[pl.ds(off, LB)], rows_v.at[0])
            pltpu.sync_copy(rows_v.at[0], agg_out.at[c, pl.ds(off, LB)])
            if with_deg:
                pltpu.sync_copy(deg_sh.at[pl.ds(off, LB)], zd_v)
                pltpu.sync_copy(zd_v, deg_out.at[c, pl.ds(off, LB)])
            return 0
        lax.fori_loop(0, TFC, wchunk, 0)

    return pl.kernel(
        body, out_type=out_type, mesh=mesh, scratch_types=scratch,
        compiler_params=pltpu.CompilerParams(use_tc_tiling_on_sc=False),
    )


_sc32 = _make_sc(32, True)
_sc16 = _make_sc(16, False)


def kernel(x, edge_index, W_self1, W_neigh1, b1, W_self2, W_neigh2, b2):
    src = edge_index[0].astype(jnp.int32)
    dst = edge_index[1].astype(jnp.int32)
    src_p = jnp.concatenate([src, jnp.zeros((EP - E,), jnp.int32)]).reshape(NW, NB, LB)
    dst_p = jnp.concatenate([dst, jnp.full((EP - E,), N, jnp.int32)]).reshape(NW, NB, LB)

    z32 = jnp.zeros((LB, 32), jnp.float32)
    z16 = jnp.zeros((LB, 16), jnp.float32)
    ones1 = jnp.ones((LB,), jnp.float32)
    z1 = jnp.zeros((LB,), jnp.float32)

    w1 = jnp.concatenate([W_self1, W_neigh1], axis=1)
    b1c = jnp.concatenate([b1, jnp.zeros((32,), jnp.float32)]).reshape(1, 64)
    w2 = jnp.concatenate([W_self2, W_neigh2], axis=1)
    b2c = jnp.concatenate([b2, jnp.zeros((16,), jnp.float32)]).reshape(1, 32)

    s1, y1 = _mm1(x, w1, b1c)
    agg1, deg = _sc32(y1, src_p, dst_p, z32, ones1, z1)
    degp = deg.reshape(NC, NP, 1)
    s2, y2 = _mid(s1, agg1, degp, w2, b2c)
    agg2 = _sc16(y2, src_p, dst_p, z16, ones1, z1)[0]
    return _out(s2, agg2, degp)


# speedup vs baseline: 11.1398x; 1.0095x over previous
import jax
import jax.numpy as jnp
from jax import lax
from jax.experimental import pallas as pl
from jax.experimental.pallas import tpu as pltpu
from jax.experimental.pallas import tpu_sc as plsc

N = 50000
NP = 51200
RB = 1000
NG = N // RB
NC = 2
NS = 16
LB = 128
K = 4
NW = NC * NS
NB = 196
NCH = NB // K
EPW = NB * LB
EP = NW * EPW
E = 800000
TPR = NP // NS
TFC = TPR // LB


def _mm1_body(x_ref, w_ref, b_ref, s_ref, y_ref):
    r = jnp.dot(x_ref[...], w_ref[...], preferred_element_type=jnp.float32) + b_ref[...]
    s_ref[...] = r[:, :32]
    y_ref[...] = r[:, 32:]


def _mid_body(s_ref, agg_ref, deg_ref, w_ref, b_ref, s2_ref, y_ref, rd_ref):
    a = agg_ref[...]
    d = deg_ref[...]
    rd = 1.0 / jnp.maximum(d[0] + d[1], 1.0)
    h = s_ref[...] + (a[0] + a[1]) * rd
    r = jnp.dot(h, w_ref[...], preferred_element_type=jnp.float32) + b_ref[...]
    s2_ref[...] = r[:, :16]
    y_ref[...] = r[:, 16:]
    rd_ref[...] = jnp.broadcast_to(rd, (RB, 16))


def _out_body(s_ref, agg_ref, rd_ref, o_ref):
    a = agg_ref[...]
    o_ref[...] = s_ref[...] + (a[0] + a[1]) * rd_ref[...]


def _mm1(x, w, b):
    return pl.pallas_call(
        _mm1_body,
        grid=(NG,),
        in_specs=[
            pl.BlockSpec((RB, 64), lambda i: (i, 0)),
            pl.BlockSpec((64, 64), lambda i: (0, 0)),
            pl.BlockSpec((1, 64), lambda i: (0, 0)),
        ],
        out_specs=[
            pl.BlockSpec((RB, 32), lambda i: (i, 0)),
            pl.BlockSpec((RB, 32), lambda i: (i, 0)),
        ],
        out_shape=[
            jax.ShapeDtypeStruct((N, 32), jnp.float32),
            jax.ShapeDtypeStruct((N, 32), jnp.float32),
        ],
    )(x, w, b)


def _mid(s1, aggp, degp, w, b):
    return pl.pallas_call(
        _mid_body,
        grid=(NG,),
        in_specs=[
            pl.BlockSpec((RB, 32), lambda i: (i, 0)),
            pl.BlockSpec((NC, RB, 32), lambda i: (0, i, 0)),
            pl.BlockSpec((NC, RB, 1), lambda i: (0, i, 0)),
            pl.BlockSpec((32, 32), lambda i: (0, 0)),
            pl.BlockSpec((1, 32), lambda i: (0, 0)),
        ],
        out_specs=[
            pl.BlockSpec((RB, 16), lambda i: (i, 0)),
            pl.BlockSpec((RB, 16), lambda i: (i, 0)),
            pl.BlockSpec((RB, 16), lambda i: (i, 0)),
        ],
        out_shape=[
            jax.ShapeDtypeStruct((N, 16), jnp.float32),
            jax.ShapeDtypeStruct((N, 16), jnp.float32),
            jax.ShapeDtypeStruct((N, 16), jnp.float32),
        ],
    )(s1, aggp, degp, w, b)


def _out(s2, aggp, rd16):
    return pl.pallas_call(
        _out_body,
        grid=(NG,),
        in_specs=[
            pl.BlockSpec((RB, 16), lambda i: (i, 0)),
            pl.BlockSpec((NC, RB, 16), lambda i: (0, i, 0)),
            pl.BlockSpec((RB, 16), lambda i: (i, 0)),
        ],
        out_specs=pl.BlockSpec((RB, 16), lambda i: (i, 0)),
        out_shape=jax.ShapeDtypeStruct((N, 16), jnp.float32),
    )(s2, aggp, rd16)


def _make_sc(F, with_deg):
    mesh = plsc.VectorSubcoreMesh(core_axis_name="c", subcore_axis_name="s")
    out_type = [jax.ShapeDtypeStruct((NC, NP, F), jnp.float32)]
    scratch = [
        pltpu.VMEM((K, LB), jnp.int32),
        pltpu.VMEM((K, LB), jnp.int32),
        pltpu.VMEM((K, LB, F), jnp.float32),
        [pltpu.SemaphoreType.DMA] * K,
        pltpu.VMEM_SHARED((NP, F), jnp.float32),
    ]
    if with_deg:
        out_type.append(jax.ShapeDtypeStruct((NC, NP), jnp.float32))
        scratch += [
            pltpu.VMEM((LB,), jnp.float32),
            pltpu.VMEM((LB,), jnp.float32),
            pltpu.VMEM_SHARED((NP,), jnp.float32),
        ]

    def body(y_hbm, src_hbm, dst_hbm, zrow_hbm, ones_hbm, z1_hbm, *rest):
        if with_deg:
            (agg_out, deg_out, src_v, dst_v, rows_v, gsem, agg_sh,
             ones_v, zd_v, deg_sh) = rest
        else:
            agg_out, src_v, dst_v, rows_v, gsem, agg_sh = rest
        c = lax.axis_index("c")
        s = lax.axis_index("s")
        w = c * NS + s
        base = s * TPR

        pltpu.sync_copy(zrow_hbm, rows_v.at[0])
        if with_deg:
            pltpu.sync_copy(ones_hbm, ones_v)
            pltpu.sync_copy(z1_hbm, zd_v)

        def zchunk(i, _):
            pltpu.sync_copy(rows_v.at[0], agg_sh.at[pl.ds(base + i * LB, LB)])
            if with_deg:
                pltpu.sync_copy(zd_v, deg_sh.at[pl.ds(base + i * LB, LB)])
            return 0
        lax.fori_loop(0, TFC, zchunk, 0)
        plsc.subcore_barrier()

        def chunk(ci, _):
            s0 = w * NB + ci * K
            pltpu.sync_copy(src_hbm.at[pl.ds(s0, K)], src_v)
            pltpu.sync_copy(dst_hbm.at[pl.ds(s0, K)], dst_v)
            gds = [pltpu.async_copy(y_hbm.at[src_v.at[b]], rows_v.at[b], gsem[b])
                   for b in range(K)]
            for b in range(K):
                gds[b].wait()
                pltpu.sync_copy(rows_v.at[b], agg_sh.at[dst_v.at[b]], add=True)
                if with_deg:
                    pltpu.sync_copy(ones_v, deg_sh.at[dst_v.at[b]], add=True)
            return 0
        lax.fori_loop(0, NCH, chunk, 0)
        plsc.subcore_barrier()

        def wchunk(i, _):
            off = base + i * LB
            pltpu.sync_copy(agg_sh.at[pl.ds(off, LB)], rows_v.at[0])
            pltpu.sync_copy(rows_v.at[0], agg_out.at[c, pl.ds(off, LB)])
            if with_deg:
                pltpu.sync_copy(deg_sh.at[pl.ds(off, LB)], zd_v)
                pltpu.sync_copy(zd_v, deg_out.at[c, pl.ds(off, LB)])
            return 0
        lax.fori_loop(0, TFC, wchunk, 0)

    return pl.kernel(
        body, out_type=out_type, mesh=mesh, scratch_types=scratch,
        compiler_params=pltpu.CompilerParams(use_tc_tiling_on_sc=False),
    )


_sc32 = _make_sc(32, True)
_sc16 = _make_sc(16, False)


def kernel(x, edge_index, W_self1, W_neigh1, b1, W_self2, W_neigh2, b2):
    src = edge_index[0].astype(jnp.int32)
    dst = edge_index[1].astype(jnp.int32)
    src_p = jnp.concatenate([src, jnp.zeros((EP - E,), jnp.int32)]).reshape(NW * NB, LB)
    dst_p = jnp.concatenate([dst, jnp.full((EP - E,), N, jnp.int32)]).reshape(NW * NB, LB)

    z32 = jnp.zeros((LB, 32), jnp.float32)
    z16 = jnp.zeros((LB, 16), jnp.float32)
    ones1 = jnp.ones((LB,), jnp.float32)
    z1 = jnp.zeros((LB,), jnp.float32)

    w1 = jnp.concatenate([W_self1, W_neigh1], axis=1)
    b1c = jnp.concatenate([b1, jnp.zeros((32,), jnp.float32)]).reshape(1, 64)
    w2 = jnp.concatenate([W_self2, W_neigh2], axis=1)
    b2c = jnp.concatenate([b2, jnp.zeros((16,), jnp.float32)]).reshape(1, 32)

    s1, y1 = _mm1(x, w1, b1c)
    agg1, deg = _sc32(y1, src_p, dst_p, z32, ones1, z1)
    degp = deg.reshape(NC, NP, 1)
    s2, y2, rd16 = _mid(s1, agg1, degp, w2, b2c)
    agg2 = _sc16(y2, src_p, dst_p, z16, ones1, z1)[0]
    return _out(s2, agg2, rd16)


# speedup vs baseline: 12.1885x; 1.0941x over previous
import jax
import jax.numpy as jnp
from jax import lax
from jax.experimental import pallas as pl
from jax.experimental.pallas import tpu as pltpu
from jax.experimental.pallas import tpu_sc as plsc

N = 50000
NP = 51200
RB = 2000
NG = N // RB
NC = 2
NS = 16
LB = 128
NW = NC * NS
NB = 196
EPW = NB * LB
EP = NW * EPW
E = 800000
TPR = NP // NS
TFC = TPR // LB


def _mm1_body(x_ref, w_ref, b_ref, s_ref, y_ref):
    r = jnp.dot(x_ref[...], w_ref[...], preferred_element_type=jnp.float32) + b_ref[...]
    s_ref[...] = r[:, :32]
    y_ref[...] = r[:, 32:]


def _mid_body(s_ref, agg_ref, deg_ref, w_ref, b_ref, s2_ref, y_ref, rd_ref):
    a = agg_ref[...]
    d = deg_ref[...]
    rd = 1.0 / jnp.maximum(d[0] + d[1], 1.0)
    h = s_ref[...] + (a[0] + a[1]) * rd
    r = jnp.dot(h, w_ref[...], preferred_element_type=jnp.float32) + b_ref[...]
    s2_ref[...] = r[:, :16]
    y_ref[...] = r[:, 16:]
    rd_ref[...] = jnp.broadcast_to(rd, (RB, 16))


def _out_body(s_ref, agg_ref, rd_ref, o_ref):
    a = agg_ref[...]
    o_ref[...] = s_ref[...] + (a[0] + a[1]) * rd_ref[...]


def _mm1(x, w, b):
    return pl.pallas_call(
        _mm1_body,
        grid=(NG,),
        in_specs=[
            pl.BlockSpec((RB, 64), lambda i: (i, 0)),
            pl.BlockSpec((64, 64), lambda i: (0, 0)),
            pl.BlockSpec((1, 64), lambda i: (0, 0)),
        ],
        out_specs=[
            pl.BlockSpec((RB, 32), lambda i: (i, 0)),
            pl.BlockSpec((RB, 32), lambda i: (i, 0)),
        ],
        out_shape=[
            jax.ShapeDtypeStruct((N, 32), jnp.float32),
            jax.ShapeDtypeStruct((N, 32), jnp.float32),
        ],
    )(x, w, b)


def _mid(s1, aggp, degp, w, b):
    return pl.pallas_call(
        _mid_body,
        grid=(NG,),
        in_specs=[
            pl.BlockSpec((RB, 32), lambda i: (i, 0)),
            pl.BlockSpec((NC, RB, 32), lambda i: (0, i, 0)),
            pl.BlockSpec((NC, RB, 1), lambda i: (0, i, 0)),
            pl.BlockSpec((32, 32), lambda i: (0, 0)),
            pl.BlockSpec((1, 32), lambda i: (0, 0)),
        ],
        out_specs=[
            pl.BlockSpec((RB, 16), lambda i: (i, 0)),
            pl.BlockSpec((RB, 16), lambda i: (i, 0)),
            pl.BlockSpec((RB, 16), lambda i: (i, 0)),
        ],
        out_shape=[
            jax.ShapeDtypeStruct((N, 16), jnp.float32),
            jax.ShapeDtypeStruct((N, 16), jnp.float32),
            jax.ShapeDtypeStruct((N, 16), jnp.float32),
        ],
    )(s1, aggp, degp, w, b)


def _out(s2, aggp, rd16):
    return pl.pallas_call(
        _out_body,
        grid=(NG,),
        in_specs=[
            pl.BlockSpec((RB, 16), lambda i: (i, 0)),
            pl.BlockSpec((NC, RB, 16), lambda i: (0, i, 0)),
            pl.BlockSpec((RB, 16), lambda i: (i, 0)),
        ],
        out_specs=pl.BlockSpec((RB, 16), lambda i: (i, 0)),
        out_shape=jax.ShapeDtypeStruct((N, 16), jnp.float32),
    )(s2, aggp, rd16)


def _make_sc(F, with_deg, K):
    NCH = NB // K
    mesh = plsc.VectorSubcoreMesh(core_axis_name="c", subcore_axis_name="s")
    out_type = [jax.ShapeDtypeStruct((NC, NP, F), jnp.float32)]
    scratch = [
        pltpu.VMEM((K, LB), jnp.int32),
        pltpu.VMEM((K, LB), jnp.int32),
        pltpu.VMEM((K, LB, F), jnp.float32),
        [pltpu.SemaphoreType.DMA] * K,
        pltpu.VMEM_SHARED((NP, F), jnp.float32),
    ]
    if with_deg:
        out_type.append(jax.ShapeDtypeStruct((NC, NP), jnp.float32))
        scratch += [
            pltpu.VMEM((LB,), jnp.float32),
            pltpu.VMEM((LB,), jnp.float32),
            pltpu.VMEM_SHARED((NP,), jnp.float32),
        ]

    def body(y_hbm, src_hbm, dst_hbm, zrow_hbm, ones_hbm, z1_hbm, *rest):
        if with_deg:
            (agg_out, deg_out, src_v, dst_v, rows_v, gsem, agg_sh,
             ones_v, zd_v, deg_sh) = rest
        else:
            agg_out, src_v, dst_v, rows_v, gsem, agg_sh = rest
        c = lax.axis_index("c")
        s = lax.axis_index("s")
        w = c * NS + s
        base = s * TPR

        pltpu.sync_copy(zrow_hbm, rows_v.at[0])
        if with_deg:
            pltpu.sync_copy(ones_hbm, ones_v)
            pltpu.sync_copy(z1_hbm, zd_v)

        def zchunk(i, _):
            pltpu.sync_copy(rows_v.at[0], agg_sh.at[pl.ds(base + i * LB, LB)])
            if with_deg:
                pltpu.sync_copy(zd_v, deg_sh.at[pl.ds(base + i * LB, LB)])
            return 0
        lax.fori_loop(0, TFC, zchunk, 0)
        plsc.subcore_barrier()

        def chunk(ci, _):
            s0 = w * NB + ci * K
            pltpu.sync_copy(src_hbm.at[pl.ds(s0, K)], src_v)
            pltpu.sync_copy(dst_hbm.at[pl.ds(s0, K)], dst_v)
            gds = [pltpu.async_copy(y_hbm.at[src_v.at[b]], rows_v.at[b], gsem[b])
                   for b in range(K)]
            for b in range(K):
                gds[b].wait()
                pltpu.sync_copy(rows_v.at[b], agg_sh.at[dst_v.at[b]], add=True)
                if with_deg:
                    pltpu.sync_copy(ones_v, deg_sh.at[dst_v.at[b]], add=True)
            return 0
        lax.fori_loop(0, NCH, chunk, 0)
        plsc.subcore_barrier()

        def wchunk(i, _):
            off = base + i * LB
            pltpu.sync_copy(agg_sh.at[pl.ds(off, LB)], rows_v.at[0])
            pltpu.sync_copy(rows_v.at[0], agg_out.at[c, pl.ds(off, LB)])
            if with_deg:
                pltpu.sync_copy(deg_sh.at[pl.ds(off, LB)], zd_v)
                pltpu.sync_copy(zd_v, deg_out.at[c, pl.ds(off, LB)])
            return 0
        lax.fori_loop(0, TFC, wchunk, 0)

    return pl.kernel(
        body, out_type=out_type, mesh=mesh, scratch_types=scratch,
        compiler_params=pltpu.CompilerParams(use_tc_tiling_on_sc=False),
    )


_sc32 = _make_sc(32, True, 4)
_sc16 = _make_sc(16, False, 7)


def kernel(x, edge_index, W_self1, W_neigh1, b1, W_self2, W_neigh2, b2):
    src = edge_index[0].astype(jnp.int32)
    dst = edge_index[1].astype(jnp.int32)
    src_p = jnp.concatenate([src, jnp.zeros((EP - E,), jnp.int32)]).reshape(NW * NB, LB)
    dst_p = jnp.concatenate([dst, jnp.full((EP - E,), N, jnp.int32)]).reshape(NW * NB, LB)

    z32 = jnp.zeros((LB, 32), jnp.float32)
    z16 = jnp.zeros((LB, 16), jnp.float32)
    ones1 = jnp.ones((LB,), jnp.float32)
    z1 = jnp.zeros((LB,), jnp.float32)

    w1 = jnp.concatenate([W_self1, W_neigh1], axis=1)
    b1c = jnp.concatenate([b1, jnp.zeros((32,), jnp.float32)]).reshape(1, 64)
    w2 = jnp.concatenate([W_self2, W_neigh2], axis=1)
    b2c = jnp.concatenate([b2, jnp.zeros((16,), jnp.float32)]).reshape(1, 32)

    s1, y1 = _mm1(x, w1, b1c)
    agg1, deg = _sc32(y1, src_p, dst_p, z32, ones1, z1)
    degp = deg.reshape(NC, NP, 1)
    s2, y2, rd16 = _mid(s1, agg1, degp, w2, b2c)
    agg2 = _sc16(y2, src_p, dst_p, z16, ones1, z1)[0]
    return _out(s2, agg2, rd16)


# speedup vs baseline: 12.6921x; 1.0413x over previous
import jax
import jax.numpy as jnp
from jax import lax
from jax.experimental import pallas as pl
from jax.experimental.pallas import tpu as pltpu
from jax.experimental.pallas import tpu_sc as plsc

N = 50000
NP = 51200
RB = 2000
NG = N // RB
NC = 2
NS = 16
LB = 128
NW = NC * NS
NB = 196
EPW = NB * LB
EP = NW * EPW
E = 800000
TPR = NP // NS
TFC = TPR // LB


def _mm1_body(x_ref, w_ref, b_ref, s_ref, y_ref):
    r = jnp.dot(x_ref[...], w_ref[...], preferred_element_type=jnp.float32) + b_ref[...]
    s_ref[...] = r[:, :32]
    y_ref[...] = r[:, 32:]


def _mid_body(s_ref, agg_ref, deg_ref, w_ref, b_ref, s2_ref, y_ref, rd_ref):
    a = agg_ref[...]
    d = deg_ref[...]
    rd = 1.0 / jnp.maximum(d[0] + d[1], 1.0)
    h = s_ref[...] + (a[0] + a[1]) * rd
    r = jnp.dot(h, w_ref[...], preferred_element_type=jnp.float32) + b_ref[...]
    s2_ref[...] = r[:, :16]
    y_ref[...] = r[:, 16:]
    rd_ref[...] = jnp.broadcast_to(rd, (RB, 16))


def _out_body(s_ref, agg_ref, rd_ref, o_ref):
    a = agg_ref[...]
    o_ref[...] = s_ref[...] + (a[0] + a[1]) * rd_ref[...]


def _mm1(x, w, b):
    return pl.pallas_call(
        _mm1_body,
        grid=(NG,),
        in_specs=[
            pl.BlockSpec((RB, 64), lambda i: (i, 0)),
            pl.BlockSpec((64, 64), lambda i: (0, 0)),
            pl.BlockSpec((1, 64), lambda i: (0, 0)),
        ],
        out_specs=[
            pl.BlockSpec((RB, 32), lambda i: (i, 0)),
            pl.BlockSpec((RB, 32), lambda i: (i, 0)),
        ],
        out_shape=[
            jax.ShapeDtypeStruct((N, 32), jnp.float32),
            jax.ShapeDtypeStruct((N, 32), jnp.float32),
        ],
    )(x, w, b)


def _mid(s1, aggp, degp, w, b):
    return pl.pallas_call(
        _mid_body,
        grid=(NG,),
        in_specs=[
            pl.BlockSpec((RB, 32), lambda i: (i, 0)),
            pl.BlockSpec((NC, RB, 32), lambda i: (0, i, 0)),
            pl.BlockSpec((NC, RB, 1), lambda i: (0, i, 0)),
            pl.BlockSpec((32, 32), lambda i: (0, 0)),
            pl.BlockSpec((1, 32), lambda i: (0, 0)),
        ],
        out_specs=[
            pl.BlockSpec((RB, 16), lambda i: (i, 0)),
            pl.BlockSpec((RB, 16), lambda i: (i, 0)),
            pl.BlockSpec((RB, 16), lambda i: (i, 0)),
        ],
        out_shape=[
            jax.ShapeDtypeStruct((N, 16), jnp.float32),
            jax.ShapeDtypeStruct((N, 16), jnp.float32),
            jax.ShapeDtypeStruct((N, 16), jnp.float32),
        ],
    )(s1, aggp, degp, w, b)


def _out(s2, aggp, rd16):
    return pl.pallas_call(
        _out_body,
        grid=(NG,),
        in_specs=[
            pl.BlockSpec((RB, 16), lambda i: (i, 0)),
            pl.BlockSpec((NC, RB, 16), lambda i: (0, i, 0)),
            pl.BlockSpec((RB, 16), lambda i: (i, 0)),
        ],
        out_specs=pl.BlockSpec((RB, 16), lambda i: (i, 0)),
        out_shape=jax.ShapeDtypeStruct((N, 16), jnp.float32),
    )(s2, aggp, rd16)


def _make_sc(F, with_deg, K):
    NCH = NB // K
    mesh = plsc.VectorSubcoreMesh(core_axis_name="c", subcore_axis_name="s")
    out_type = [jax.ShapeDtypeStruct((NC, NP, F), jnp.float32)]
    scratch = [
        pltpu.VMEM((K, LB), jnp.int32),
        pltpu.VMEM((K, LB), jnp.int32),
        pltpu.VMEM((K, LB, F), jnp.float32),
        [pltpu.SemaphoreType.DMA] * K,
        pltpu.SemaphoreType.DMA,
        pltpu.VMEM_SHARED((NP, F), jnp.float32),
    ]
    if with_deg:
        out_type.append(jax.ShapeDtypeStruct((NC, NP), jnp.float32))
        scratch += [
            pltpu.VMEM((LB,), jnp.float32),
            pltpu.VMEM((LB,), jnp.float32),
            pltpu.VMEM_SHARED((NP,), jnp.float32),
        ]

    def body(y_hbm, src_hbm, dst_hbm, zrow_hbm, ones_hbm, z1_hbm, *rest):
        if with_deg:
            (agg_out, deg_out, src_v, dst_v, rows_v, gsem, ssem, agg_sh,
             ones_v, zd_v, deg_sh) = rest
        else:
            agg_out, src_v, dst_v, rows_v, gsem, ssem, agg_sh = rest
        c = lax.axis_index("c")
        s = lax.axis_index("s")
        w = c * NS + s
        base = s * TPR

        pltpu.sync_copy(zrow_hbm, rows_v.at[0])
        if with_deg:
            pltpu.sync_copy(ones_hbm, ones_v)
            pltpu.sync_copy(z1_hbm, zd_v)

        def zchunk(i, _):
            pltpu.sync_copy(rows_v.at[0], agg_sh.at[pl.ds(base + i * LB, LB)])
            if with_deg:
                pltpu.sync_copy(zd_v, deg_sh.at[pl.ds(base + i * LB, LB)])
            return 0
        lax.fori_loop(0, TFC, zchunk, 0)
        plsc.subcore_barrier()

        def chunk(ci, _):
            s0 = w * NB + ci * K
            pltpu.sync_copy(src_hbm.at[pl.ds(s0, K)], src_v)
            pltpu.sync_copy(dst_hbm.at[pl.ds(s0, K)], dst_v)
            gds = [pltpu.async_copy(y_hbm.at[src_v.at[b]], rows_v.at[b], gsem[b])
                   for b in range(K)]
            sds = []
            for b in range(K):
                gds[b].wait()
                sds.append(pltpu.async_copy(
                    rows_v.at[b], agg_sh.at[dst_v.at[b]], ssem, add=True))
                if with_deg:
                    pltpu.sync_copy(ones_v, deg_sh.at[dst_v.at[b]], add=True)
            for dsc in sds:
                dsc.wait()
            return 0
        lax.fori_loop(0, NCH, chunk, 0)
        plsc.subcore_barrier()

        def wchunk(i, _):
            off = base + i * LB
            pltpu.sync_copy(agg_sh.at[pl.ds(off, LB)], rows_v.at[0])
            pltpu.sync_copy(rows_v.at[0], agg_out.at[c, pl.ds(off, LB)])
            if with_deg:
                pltpu.sync_copy(deg_sh.at[pl.ds(off, LB)], zd_v)
                pltpu.sync_copy(zd_v, deg_out.at[c, pl.ds(off, LB)])
            return 0
        lax.fori_loop(0, TFC, wchunk, 0)

    return pl.kernel(
        body, out_type=out_type, mesh=mesh, scratch_types=scratch,
        compiler_params=pltpu.CompilerParams(use_tc_tiling_on_sc=False),
    )


_sc32 = _make_sc(32, True, 4)
_sc16 = _make_sc(16, False, 7)


def kernel(x, edge_index, W_self1, W_neigh1, b1, W_self2, W_neigh2, b2):
    src = edge_index[0].astype(jnp.int32)
    dst = edge_index[1].astype(jnp.int32)
    src_p = jnp.concatenate([src, jnp.zeros((EP - E,), jnp.int32)]).reshape(NW * NB, LB)
    dst_p = jnp.concatenate([dst, jnp.full((EP - E,), N, jnp.int32)]).reshape(NW * NB, LB)

    z32 = jnp.zeros((LB, 32), jnp.float32)
    z16 = jnp.zeros((LB, 16), jnp.float32)
    ones1 = jnp.ones((LB,), jnp.float32)
    z1 = jnp.zeros((LB,), jnp.float32)

    w1 = jnp.concatenate([W_self1, W_neigh1], axis=1)
    b1c = jnp.concatenate([b1, jnp.zeros((32,), jnp.float32)]).reshape(1, 64)
    w2 = jnp.concatenate([W_self2, W_neigh2], axis=1)
    b2c = jnp.concatenate([b2, jnp.zeros((16,), jnp.float32)]).reshape(1, 32)

    s1, y1 = _mm1(x, w1, b1c)
    agg1, deg = _sc32(y1, src_p, dst_p, z32, ones1, z1)
    degp = deg.reshape(NC, NP, 1)
    s2, y2, rd16 = _mid(s1, agg1, degp, w2, b2c)
    agg2 = _sc16(y2, src_p, dst_p, z16, ones1, z1)[0]
    return _out(s2, agg2, rd16)
